# Initial kernel scaffold; baseline (speedup 1.0000x reference)
import jax
import jax.numpy as jnp
from jax.experimental import pallas as pl


def kernel(word_ids, table, gamma, beta):
    raise NotImplementedError("write your pallas kernel here")


# speedup vs baseline: 1.8416x; 1.8416x over previous
import dataclasses
import functools

import jax
import jax.numpy as jnp
from jax import lax
from jax.experimental import pallas as pl
from jax.experimental.pallas import tpu as pltpu
from jax.experimental.pallas import tpu_sc as plsc

NUM_WORD = 1000000
EMB_DIM = 64
BATCH = 16384
HIST = 50
EPS = 1e-5

NC = 2
NS = 16
NW = NC * NS
LANES = 16
NREG = EMB_DIM // LANES

B_PER_W = BATCH // NW
CB = 8
IDX_PER_CHUNK = CB * HIST
N_CHUNKS = B_PER_W // CB
GATHER_SLICE = 80
N_SLICES = IDX_PER_CHUNK // GATHER_SLICE

_MAGIC = 0x5F3759DF


def _fast_rsqrt(v):
    bi = lax.bitcast_convert_type(v, jnp.int32)
    bi = jnp.full((LANES,), _MAGIC, dtype=jnp.int32) - lax.shift_right_arithmetic(bi, 1)
    y = lax.bitcast_convert_type(bi, jnp.float32)
    half = 0.5 * v
    for _ in range(3):
        y = y * (1.5 - half * y * y)
    return y


def _emb_body(idx_hbm, table_hbm, gamma_hbm, beta_hbm, out_hbm,
              idx_v, rows_v, out_v, gb_v, t0_v, sem):
    wid = lax.axis_index("s") * NC + lax.axis_index("c")

    pltpu.sync_copy(gamma_hbm, gb_v.at[0])
    pltpu.sync_copy(beta_hbm, gb_v.at[1])
    pltpu.sync_copy(table_hbm.at[pl.ds(0, 1)], t0_v)

    g = [gb_v[0, pl.ds(k * LANES, LANES)] for k in range(NREG)]
    bt = [gb_v[1, pl.ds(k * LANES, LANES)] for k in range(NREG)]
    t0 = [t0_v[0, pl.ds(k * LANES, LANES)] for k in range(NREG)]
    lane = lax.broadcasted_iota(jnp.int32, (LANES,), 0)

    idx_base = wid * (B_PER_W * HIST)
    row_base = wid * B_PER_W

    @pl.loop(0, N_CHUNKS)
    def _chunk(m):
        ioff = idx_base + m * IDX_PER_CHUNK
        pltpu.sync_copy(idx_hbm.at[pl.ds(ioff, IDX_PER_CHUNK)], idx_v)
        cps = [
            pltpu.async_copy(
                table_hbm.at[idx_v.at[pl.ds(k * GATHER_SLICE, GATHER_SLICE)]],
                rows_v.at[pl.ds(k * GATHER_SLICE, GATHER_SLICE)],
                sem,
            )
            for k in range(N_SLICES)
        ]
        for c in cps:
            c.wait()

        for r in range(CB):
            roff = r * HIST
            acc = [rows_v[roff, pl.ds(k * LANES, LANES)] for k in range(NREG)]
            for j in range(1, HIST):
                for k in range(NREG):
                    acc[k] = acc[k] + rows_v[roff + j, pl.ds(k * LANES, LANES)]

            z0 = idx_v[pl.ds(roff, LANES)] == 0
            z1 = idx_v[pl.ds(roff + 16, LANES)] == 0
            z2 = idx_v[pl.ds(roff + 32, LANES)] == 0
            z3 = (idx_v[pl.ds(roff + 34, LANES)] == 0) & (lane >= 14)
            cnt = (
                plsc.all_reduce_population_count(z0)
                + plsc.all_reduce_population_count(z1)
                + plsc.all_reduce_population_count(z2)
                + plsc.all_reduce_population_count(z3)
            ).astype(jnp.float32)

            a = [(acc[k] - cnt * t0[k]) * (1.0 / HIST) for k in range(NREG)]

            s = a[0] + a[1] + a[2] + a[3]
            q = a[0] * a[0] + a[1] * a[1] + a[2] * a[2] + a[3] * a[3]
            tot = jnp.sum(s)
            sq = jnp.sum(q)
            mu = tot * (1.0 / EMB_DIM)
            var = sq * (1.0 / EMB_DIM) - mu * mu

            vv = jnp.full((LANES,), var + EPS, dtype=jnp.float32)
            inv = _fast_rsqrt(vv)
            mub = jnp.full((LANES,), mu, dtype=jnp.float32)
            for k in range(NREG):
                out_v[r, pl.ds(k * LANES, LANES)] = (a[k] - mub) * inv * g[k] + bt[k]

        pltpu.sync_copy(out_v, out_hbm.at[pl.ds(row_base + m * CB, CB)])


@jax.jit
def _emb_kernel(idx_flat, table, gamma, beta):
    mesh = plsc.VectorSubcoreMesh(core_axis_name="c", subcore_axis_name="s")
    cp = pltpu.CompilerParams()
    if "needs_layout_passes" in pltpu.CompilerParams.__dataclass_fields__:
        cp = dataclasses.replace(cp, needs_layout_passes=False)
    if "use_tc_tiling_on_sc" in pltpu.CompilerParams.__dataclass_fields__:
        cp = dataclasses.replace(cp, use_tc_tiling_on_sc=False)
    return pl.kernel(
        _emb_body,
        out_type=jax.ShapeDtypeStruct((BATCH, EMB_DIM), jnp.float32),
        mesh=mesh,
        scratch_types=[
            pltpu.VMEM((IDX_PER_CHUNK,), jnp.int32),
            pltpu.VMEM((IDX_PER_CHUNK, EMB_DIM), jnp.float32),
            pltpu.VMEM((CB, EMB_DIM), jnp.float32),
            pltpu.VMEM((2, EMB_DIM), jnp.float32),
            pltpu.VMEM((1, EMB_DIM), jnp.float32),
            pltpu.SemaphoreType.DMA,
        ],
        compiler_params=cp,
    )(idx_flat, table, gamma, beta)


def kernel(word_ids, table, gamma, beta):
    idx_flat = word_ids.reshape(-1).astype(jnp.int32)
    return _emb_kernel(idx_flat, table, gamma, beta)


# speedup vs baseline: 2.7215x; 1.4778x over previous
import dataclasses
import functools

import jax
import jax.numpy as jnp
from jax import lax
from jax.experimental import pallas as pl
from jax.experimental.pallas import tpu as pltpu
from jax.experimental.pallas import tpu_sc as plsc

NUM_WORD = 1000000
EMB_DIM = 64
BATCH = 16384
HIST = 50
EPS = 1e-5

NC = 2
NS = 16
NW = NC * NS
LANES = 16
NREG = EMB_DIM // LANES

B_PER_W = BATCH // NW
CB = 8
IDX_PER_CHUNK = CB * HIST
N_CHUNKS = B_PER_W // CB
GATHER_SLICE = 80
N_SLICES = IDX_PER_CHUNK // GATHER_SLICE

_MAGIC = 0x5F3759DF


def _fast_rsqrt(v):
    bi = lax.bitcast_convert_type(v, jnp.int32)
    bi = jnp.full((LANES,), _MAGIC, dtype=jnp.int32) - lax.shift_right_arithmetic(bi, 1)
    y = lax.bitcast_convert_type(bi, jnp.float32)
    half = 0.5 * v
    for _ in range(3):
        y = y * (1.5 - half * y * y)
    return y


def _emb_body(idx_hbm, table_hbm, gamma_hbm, beta_hbm, out_hbm,
              idx_v, rows_v, out_v, gb_v, t0_v, isem, gsem):
    wid = lax.axis_index("s") * NC + lax.axis_index("c")

    pltpu.sync_copy(gamma_hbm, gb_v.at[0])
    pltpu.sync_copy(beta_hbm, gb_v.at[1])
    pltpu.sync_copy(table_hbm.at[pl.ds(0, 1)], t0_v)

    g = [gb_v[0, pl.ds(k * LANES, LANES)] for k in range(NREG)]
    bt = [gb_v[1, pl.ds(k * LANES, LANES)] for k in range(NREG)]
    t0 = [t0_v[0, pl.ds(k * LANES, LANES)] for k in range(NREG)]
    lane = lax.broadcasted_iota(jnp.int32, (LANES,), 0)

    idx_base = wid * (B_PER_W * HIST)
    row_base = wid * B_PER_W

    def issue_idx(m, slot):
        pltpu.async_copy(
            idx_hbm.at[pl.ds(idx_base + m * IDX_PER_CHUNK, IDX_PER_CHUNK)],
            idx_v.at[slot], isem.at[slot])

    def wait_idx(slot):
        pltpu.make_async_copy(
            idx_hbm.at[pl.ds(0, IDX_PER_CHUNK)],
            idx_v.at[slot], isem.at[slot]).wait()

    def issue_gathers(slot):
        for k in range(N_SLICES):
            pltpu.async_copy(
                table_hbm.at[idx_v.at[slot].at[pl.ds(k * GATHER_SLICE, GATHER_SLICE)]],
                rows_v.at[slot].at[pl.ds(k * GATHER_SLICE, GATHER_SLICE)],
                gsem.at[slot])

    def wait_gathers(slot):
        pltpu.make_async_copy(
            table_hbm.at[pl.ds(0, IDX_PER_CHUNK)],
            rows_v.at[slot], gsem.at[slot]).wait()

    def compute(m, slot):
        rows = rows_v.at[slot]
        idx = idx_v.at[slot]

        @pl.loop(0, CB)
        def _row(r):
            roff = r * HIST
            acc = [rows[roff, pl.ds(k * LANES, LANES)] for k in range(NREG)]
            for j in range(1, HIST):
                for k in range(NREG):
                    acc[k] = acc[k] + rows[roff + j, pl.ds(k * LANES, LANES)]

            z0 = idx[pl.ds(roff, LANES)] == 0
            z1 = idx[pl.ds(roff + 16, LANES)] == 0
            z2 = idx[pl.ds(roff + 32, LANES)] == 0
            z3 = (idx[pl.ds(roff + 34, LANES)] == 0) & (lane >= 14)
            cnt = (
                plsc.all_reduce_population_count(z0)
                + plsc.all_reduce_population_count(z1)
                + plsc.all_reduce_population_count(z2)
                + plsc.all_reduce_population_count(z3)
            ).astype(jnp.float32)

            a = [(acc[k] - cnt * t0[k]) * (1.0 / HIST) for k in range(NREG)]

            s = a[0] + a[1] + a[2] + a[3]
            q = a[0] * a[0] + a[1] * a[1] + a[2] * a[2] + a[3] * a[3]
            mu = jnp.sum(s) * (1.0 / EMB_DIM)
            var = jnp.sum(q) * (1.0 / EMB_DIM) - mu * mu

            inv = _fast_rsqrt(jnp.full((LANES,), var + EPS, dtype=jnp.float32))
            mub = jnp.full((LANES,), mu, dtype=jnp.float32)
            orow = m * CB + r
            for k in range(NREG):
                out_v[orow, pl.ds(k * LANES, LANES)] = (a[k] - mub) * inv * g[k] + bt[k]

    issue_idx(0, 0)
    issue_idx(1, 1)
    wait_idx(0)
    issue_gathers(0)

    @pl.loop(0, (N_CHUNKS - 2) // 2)
    def _pair(t):
        for p in range(2):
            m = 2 * t + p
            wait_gathers(p)
            wait_idx(1 - p)
            issue_gathers(1 - p)
            issue_idx(m + 2, p)
            compute(m, p)

    wait_gathers(0)
    wait_idx(1)
    issue_gathers(1)
    compute(N_CHUNKS - 2, 0)
    wait_gathers(1)
    compute(N_CHUNKS - 1, 1)

    pltpu.sync_copy(out_v, out_hbm.at[pl.ds(row_base, B_PER_W)])


@jax.jit
def _emb_kernel(idx_flat, table, gamma, beta):
    mesh = plsc.VectorSubcoreMesh(core_axis_name="c", subcore_axis_name="s")
    cp = pltpu.CompilerParams()
    if "needs_layout_passes" in pltpu.CompilerParams.__dataclass_fields__:
        cp = dataclasses.replace(cp, needs_layout_passes=False)
    if "use_tc_tiling_on_sc" in pltpu.CompilerParams.__dataclass_fields__:
        cp = dataclasses.replace(cp, use_tc_tiling_on_sc=False)
    return pl.kernel(
        _emb_body,
        out_type=jax.ShapeDtypeStruct((BATCH, EMB_DIM), jnp.float32),
        mesh=mesh,
        scratch_types=[
            pltpu.VMEM((2, IDX_PER_CHUNK), jnp.int32),
            pltpu.VMEM((2, IDX_PER_CHUNK, EMB_DIM), jnp.float32),
            pltpu.VMEM((B_PER_W, EMB_DIM), jnp.float32),
            pltpu.VMEM((2, EMB_DIM), jnp.float32),
            pltpu.VMEM((1, EMB_DIM), jnp.float32),
            pltpu.SemaphoreType.DMA((2,)),
            pltpu.SemaphoreType.DMA((2,)),
        ],
        compiler_params=cp,
    )(idx_flat, table, gamma, beta)


def kernel(word_ids, table, gamma, beta):
    idx_flat = word_ids.reshape(-1).astype(jnp.int32)
    return _emb_kernel(idx_flat, table, gamma, beta)
